# baseline (speedup 1.0000x reference)
import functools

import jax
import jax.numpy as jnp
from jax import lax
from jax.experimental import pallas as pl
from jax.experimental.pallas import tpu as pltpu
from jax.experimental.pallas import tpu_sc as plsc

_VOCAB = 100000
_HID = 64
_B = 1024
_VBLK = 2048
_NV = (_VOCAB + _VBLK - 1) // _VBLK


def _build_sc_gather():
    info = plsc.get_sparse_core_info()
    nc, ns = info.num_cores, info.num_subcores
    nw = nc * ns
    b_per_w = _B // nw
    mesh = plsc.VectorSubcoreMesh(core_axis_name="c", subcore_axis_name="s")

    @functools.partial(
        pl.kernel,
        mesh=mesh,
        out_type=jax.ShapeDtypeStruct((_B, 128), jnp.float32),
        scratch_types=[
            pltpu.VMEM((b_per_w,), jnp.int32),
            pltpu.VMEM((b_per_w, 128), jnp.float32),
            pltpu.SemaphoreType.DMA,
        ],
    )
    def gather_rows(table_hbm, idx_hbm, out_hbm, idx_v, rows_v, sem):
        wid = lax.axis_index("s") * nc + lax.axis_index("c")
        base = wid * b_per_w
        pltpu.sync_copy(idx_hbm.at[pl.ds(base, b_per_w)], idx_v)
        pltpu.async_copy(table_hbm.at[idx_v], rows_v, sem).wait()
        pltpu.sync_copy(rows_v, out_hbm.at[pl.ds(base, b_per_w)])

    return gather_rows


_sc_gather_cache = []


def _get_sc_gather():
    if not _sc_gather_cache:
        _sc_gather_cache.append(_build_sc_gather())
    return _sc_gather_cache[0]


def _tc_body(x_ref, w_ref, b_ref, lbl_ref, logits_ref, loss_ref, m_ref, s_ref, ll_ref):
    j = pl.program_id(0)

    @pl.when(j == 0)
    def _init():
        m_ref[...] = jnp.full((_B, 1), -jnp.inf, jnp.float32)
        s_ref[...] = jnp.zeros((_B, 1), jnp.float32)
        ll_ref[...] = jnp.zeros((_B, 1), jnp.float32)

    logits = (
        jnp.dot(x_ref[:, :_HID], w_ref[...], preferred_element_type=jnp.float32)
        + b_ref[...]
    )
    logits_ref[...] = logits

    col = j * _VBLK + lax.broadcasted_iota(jnp.int32, (_B, _VBLK), 1)
    lm = jnp.where(col < _VOCAB, logits, -jnp.inf)
    m_old = m_ref[...]
    m_new = jnp.maximum(m_old, jnp.max(lm, axis=1, keepdims=True))
    s_ref[...] = s_ref[...] * jnp.exp(m_old - m_new) + jnp.sum(
        jnp.exp(lm - m_new), axis=1, keepdims=True
    )
    m_ref[...] = m_new
    ll_ref[...] = ll_ref[...] + jnp.sum(
        jnp.where(col == lbl_ref[...], lm, 0.0), axis=1, keepdims=True
    )

    @pl.when(j == _NV - 1)
    def _finish():
        lse = m_ref[...] + jnp.log(s_ref[...])
        loss_ref[...] = (jnp.sum(lse - ll_ref[...]) * (1.0 / _B)).reshape(1, 1)


_tc_call = pl.pallas_call(
    _tc_body,
    grid=(_NV,),
    in_specs=[
        pl.BlockSpec((_B, 128), lambda j: (0, 0)),
        pl.BlockSpec((_HID, _VBLK), lambda j: (0, j)),
        pl.BlockSpec((1, _VBLK), lambda j: (0, j)),
        pl.BlockSpec((_B, 1), lambda j: (0, 0)),
    ],
    out_specs=[
        pl.BlockSpec((_B, _VBLK), lambda j: (0, j)),
        pl.BlockSpec((1, 1), lambda j: (0, 0)),
    ],
    out_shape=[
        jax.ShapeDtypeStruct((_B, _VOCAB), jnp.float32),
        jax.ShapeDtypeStruct((1, 1), jnp.float32),
    ],
    scratch_shapes=[
        pltpu.VMEM((_B, 1), jnp.float32),
        pltpu.VMEM((_B, 1), jnp.float32),
        pltpu.VMEM((_B, 1), jnp.float32),
    ],
    compiler_params=pltpu.CompilerParams(
        dimension_semantics=("arbitrary",),
    ),
)


def kernel(input_ids, labels, emb, W, b):
    ids = input_ids.astype(jnp.int32)
    embp = jnp.pad(emb, ((0, 0), (0, 128 - _HID)))
    x = _get_sc_gather()(embp, ids)
    b2 = b.reshape(1, _VOCAB)
    lbl = labels.astype(jnp.int32).reshape(_B, 1)
    logits, loss = _tc_call(x, W, b2, lbl)
    return loss[0, 0], logits


# speedup vs baseline: 1.8941x; 1.8941x over previous
import functools

import jax
import jax.numpy as jnp
from jax import lax
from jax.experimental import pallas as pl
from jax.experimental.pallas import tpu as pltpu
from jax.experimental.pallas import tpu_sc as plsc

_VOCAB = 100000
_HID = 64
_B = 1024
_VBLK = 2048
_NV = (_VOCAB + _VBLK - 1) // _VBLK


def _build_sc_gather():
    info = plsc.get_sparse_core_info()
    nc, ns = info.num_cores, info.num_subcores
    nw = nc * ns
    b_per_w = _B // nw
    mesh = plsc.VectorSubcoreMesh(core_axis_name="c", subcore_axis_name="s")

    @functools.partial(
        pl.kernel,
        mesh=mesh,
        out_type=jax.ShapeDtypeStruct((_B, 128), jnp.float32),
        scratch_types=[
            pltpu.VMEM((b_per_w,), jnp.int32),
            pltpu.VMEM((b_per_w, 128), jnp.float32),
            pltpu.SemaphoreType.DMA,
        ],
    )
    def gather_rows(table_hbm, idx_hbm, out_hbm, idx_v, rows_v, sem):
        wid = lax.axis_index("s") * nc + lax.axis_index("c")
        base = wid * b_per_w
        pltpu.sync_copy(idx_hbm.at[pl.ds(base, b_per_w)], idx_v)
        pltpu.async_copy(table_hbm.at[idx_v], rows_v, sem).wait()
        pltpu.sync_copy(rows_v, out_hbm.at[pl.ds(base, b_per_w)])

    return gather_rows


_sc_gather_cache = []


def _get_sc_gather():
    if not _sc_gather_cache:
        _sc_gather_cache.append(_build_sc_gather())
    return _sc_gather_cache[0]


def _tc_body(x_ref, w_ref, b_ref, lbl_ref, logits_ref, loss_ref, m_ref, s_ref, ll_ref):
    j = pl.program_id(0)

    @pl.when(j == 0)
    def _init():
        m_ref[...] = jnp.full((1, _B), -jnp.inf, jnp.float32)
        s_ref[...] = jnp.zeros((1, _B), jnp.float32)
        ll_ref[...] = jnp.zeros((1, _B), jnp.float32)

    logits = (
        lax.dot_general(
            w_ref[...],
            x_ref[:, :_HID],
            (((0,), (1,)), ((), ())),
            preferred_element_type=jnp.float32,
        )
        + b_ref[...]
    )
    logits_ref[...] = logits

    row = j * _VBLK + lax.broadcasted_iota(jnp.int32, (_VBLK, _B), 0)
    lm = jnp.where(row < _VOCAB, logits, -jnp.inf)
    m_old = m_ref[...]
    m_new = jnp.maximum(m_old, jnp.max(lm, axis=0, keepdims=True))
    s_ref[...] = s_ref[...] * jnp.exp(m_old - m_new) + jnp.sum(
        jnp.exp(lm - m_new), axis=0, keepdims=True
    )
    m_ref[...] = m_new
    ll_ref[...] = ll_ref[...] + jnp.sum(
        jnp.where(row == lbl_ref[...], lm, 0.0), axis=0, keepdims=True
    )

    @pl.when(j == _NV - 1)
    def _finish():
        lse = m_ref[...] + jnp.log(s_ref[...])
        loss_ref[...] = (jnp.sum(lse - ll_ref[...]) * (1.0 / _B)).reshape(1, 1)


_tc_call = pl.pallas_call(
    _tc_body,
    grid=(_NV,),
    in_specs=[
        pl.BlockSpec((_B, 128), lambda j: (0, 0)),
        pl.BlockSpec((_HID, _VBLK), lambda j: (0, j)),
        pl.BlockSpec((_VBLK, 1), lambda j: (j, 0)),
        pl.BlockSpec((1, _B), lambda j: (0, 0)),
    ],
    out_specs=[
        pl.BlockSpec((_VBLK, _B), lambda j: (j, 0)),
        pl.BlockSpec((1, 1), lambda j: (0, 0)),
    ],
    out_shape=[
        jax.ShapeDtypeStruct((_VOCAB, _B), jnp.float32),
        jax.ShapeDtypeStruct((1, 1), jnp.float32),
    ],
    scratch_shapes=[
        pltpu.VMEM((1, _B), jnp.float32),
        pltpu.VMEM((1, _B), jnp.float32),
        pltpu.VMEM((1, _B), jnp.float32),
    ],
    compiler_params=pltpu.CompilerParams(
        dimension_semantics=("arbitrary",),
    ),
)


def kernel(input_ids, labels, emb, W, b):
    ids = input_ids.astype(jnp.int32)
    embp = jnp.pad(emb, ((0, 0), (0, 128 - _HID)))
    x = _get_sc_gather()(embp, ids)
    b2 = b.reshape(_VOCAB, 1)
    lbl = labels.astype(jnp.int32).reshape(1, _B)
    logits_t, loss = _tc_call(x, W, b2, lbl)
    return loss[0, 0], logits_t.T


# speedup vs baseline: 2.1542x; 1.1373x over previous
import functools

import jax
import jax.numpy as jnp
from jax import lax
from jax.experimental import pallas as pl
from jax.experimental.pallas import tpu as pltpu
from jax.experimental.pallas import tpu_sc as plsc

_VOCAB = 100000
_HID = 64
_B = 1024
_VBLK = 2048
_NV = (_VOCAB + _VBLK - 1) // _VBLK


def _build_sc_gather():
    info = plsc.get_sparse_core_info()
    nc, ns = info.num_cores, info.num_subcores
    nw = nc * ns
    b_per_w = _B // nw
    mesh = plsc.VectorSubcoreMesh(core_axis_name="c", subcore_axis_name="s")

    @functools.partial(
        pl.kernel,
        mesh=mesh,
        out_type=jax.ShapeDtypeStruct((_B, 128), jnp.float32),
        scratch_types=[
            pltpu.VMEM((b_per_w,), jnp.int32),
            pltpu.VMEM((b_per_w, 128), jnp.float32),
            pltpu.SemaphoreType.DMA,
        ],
    )
    def gather_rows(table_hbm, idx_hbm, out_hbm, idx_v, rows_v, sem):
        wid = lax.axis_index("s") * nc + lax.axis_index("c")
        base = wid * b_per_w
        pltpu.sync_copy(idx_hbm.at[pl.ds(base, b_per_w)], idx_v)
        pltpu.async_copy(table_hbm.at[idx_v], rows_v, sem).wait()
        pltpu.sync_copy(rows_v, out_hbm.at[pl.ds(base, b_per_w)])

    return gather_rows


_sc_gather_cache = []


def _get_sc_gather():
    if not _sc_gather_cache:
        _sc_gather_cache.append(_build_sc_gather())
    return _sc_gather_cache[0]


def _tc_body(x_ref, w_ref, b_ref, lbl_ref, par_ref, logits_ref, loss_ref,
             m_ref, s_ref, ll_ref):
    j = pl.program_id(0)

    @pl.when(j == 0)
    def _init():
        m_ref[...] = jnp.full((1, _B), -jnp.inf, jnp.float32)
        s_ref[...] = jnp.zeros((1, _B), jnp.float32)
        ll_ref[...] = jnp.zeros((1, _B), jnp.float32)

    x_sel = jnp.where(par_ref[...] != 0, x_ref[:, _HID:2 * _HID], x_ref[:, :_HID])
    x_aug = jnp.concatenate([x_sel, jnp.ones((_B, 1), jnp.float32)], axis=1)
    w_aug = jnp.concatenate([w_ref[...], b_ref[...]], axis=0)
    logits = lax.dot_general(
        w_aug,
        x_aug,
        (((0,), (1,)), ((), ())),
        preferred_element_type=jnp.float32,
    )
    logits_ref[...] = logits

    row = j * _VBLK + lax.broadcasted_iota(jnp.int32, (_VBLK, _B), 0)
    lm = jnp.where(row < _VOCAB, logits, -jnp.inf)
    m_old = m_ref[...]
    m_new = jnp.maximum(m_old, jnp.max(lm, axis=0, keepdims=True))
    s_ref[...] = s_ref[...] * jnp.exp(m_old - m_new) + jnp.sum(
        jnp.exp(lm - m_new), axis=0, keepdims=True
    )
    m_ref[...] = m_new
    ll_ref[...] = ll_ref[...] + jnp.sum(
        jnp.where(row == lbl_ref[...], lm, 0.0), axis=0, keepdims=True
    )

    @pl.when(j == _NV - 1)
    def _finish():
        lse = m_ref[...] + jnp.log(s_ref[...])
        loss_ref[...] = (jnp.sum(lse - ll_ref[...]) * (1.0 / _B)).reshape(1, 1)


_tc_call = pl.pallas_call(
    _tc_body,
    grid=(_NV,),
    in_specs=[
        pl.BlockSpec((_B, 128), lambda j: (0, 0)),
        pl.BlockSpec((_HID, _VBLK), lambda j: (0, j)),
        pl.BlockSpec((1, _VBLK), lambda j: (0, j)),
        pl.BlockSpec((1, _B), lambda j: (0, 0)),
        pl.BlockSpec((_B, 1), lambda j: (0, 0)),
    ],
    out_specs=[
        pl.BlockSpec((_VBLK, _B), lambda j: (j, 0)),
        pl.BlockSpec((1, 1), lambda j: (0, 0)),
    ],
    out_shape=[
        jax.ShapeDtypeStruct((_VOCAB, _B), jnp.float32),
        jax.ShapeDtypeStruct((1, 1), jnp.float32),
    ],
    scratch_shapes=[
        pltpu.VMEM((1, _B), jnp.float32),
        pltpu.VMEM((1, _B), jnp.float32),
        pltpu.VMEM((1, _B), jnp.float32),
    ],
    compiler_params=pltpu.CompilerParams(
        dimension_semantics=("arbitrary",),
    ),
)


def kernel(input_ids, labels, emb, W, b):
    ids = input_ids.astype(jnp.int32)
    emb2 = emb.reshape(_VOCAB // 2, 2 * _HID)
    x = _get_sc_gather()(emb2, ids >> 1)
    par = (ids & 1).reshape(_B, 1)
    b2 = b.reshape(1, _VOCAB)
    lbl = labels.astype(jnp.int32).reshape(1, _B)
    logits_t, loss = _tc_call(x, W, b2, lbl, par)
    return loss[0, 0], logits_t.T


# speedup vs baseline: 2.2621x; 1.0501x over previous
import functools

import jax
import jax.numpy as jnp
from jax import lax
from jax.experimental import pallas as pl
from jax.experimental.pallas import tpu as pltpu
from jax.experimental.pallas import tpu_sc as plsc

_VOCAB = 100000
_HID = 64
_B = 1024
_VBLK = 2048
_NV = (_VOCAB + _VBLK - 1) // _VBLK


def _build_sc_gather():
    info = plsc.get_sparse_core_info()
    nc, ns = info.num_cores, info.num_subcores
    nw = nc * ns
    b_per_w = _B // nw
    mesh = plsc.VectorSubcoreMesh(core_axis_name="c", subcore_axis_name="s")

    @functools.partial(
        pl.kernel,
        mesh=mesh,
        out_type=jax.ShapeDtypeStruct((_B, _HID), jnp.float32),
        scratch_types=[
            pltpu.VMEM((b_per_w,), jnp.int32),
            pltpu.VMEM((b_per_w, _HID), jnp.float32),
            pltpu.SemaphoreType.DMA,
        ],
        compiler_params=pltpu.CompilerParams(use_tc_tiling_on_sc=False),
    )
    def gather_rows(table_hbm, idx_hbm, out_hbm, idx_v, rows_v, sem):
        wid = lax.axis_index("s") * nc + lax.axis_index("c")
        base = wid * b_per_w
        pltpu.sync_copy(idx_hbm.at[pl.ds(base, b_per_w)], idx_v)
        pltpu.async_copy(table_hbm.at[idx_v], rows_v, sem).wait()
        pltpu.sync_copy(rows_v, out_hbm.at[pl.ds(base, b_per_w)])

    return gather_rows


_sc_gather_cache = []


def _get_sc_gather():
    if not _sc_gather_cache:
        _sc_gather_cache.append(_build_sc_gather())
    return _sc_gather_cache[0]


def _tc_body(x_ref, w_ref, b_ref, lbl_ref, logits_ref, loss_ref,
             m_ref, s_ref, ll_ref):
    j = pl.program_id(0)

    @pl.when(j == 0)
    def _init():
        m_ref[...] = jnp.full((1, _B), -jnp.inf, jnp.float32)
        s_ref[...] = jnp.zeros((1, _B), jnp.float32)
        ll_ref[...] = jnp.zeros((1, _B), jnp.float32)

    x_aug = jnp.concatenate([x_ref[...], jnp.ones((_B, 1), jnp.float32)], axis=1)
    w_aug = jnp.concatenate([w_ref[...], b_ref[...]], axis=0)
    logits = lax.dot_general(
        w_aug,
        x_aug,
        (((0,), (1,)), ((), ())),
        preferred_element_type=jnp.float32,
    )
    logits_ref[...] = logits

    row = j * _VBLK + lax.broadcasted_iota(jnp.int32, (_VBLK, _B), 0)
    lm = jnp.where(row < _VOCAB, logits, -jnp.inf)
    m_old = m_ref[...]
    m_new = jnp.maximum(m_old, jnp.max(lm, axis=0, keepdims=True))
    s_ref[...] = s_ref[...] * jnp.exp(m_old - m_new) + jnp.sum(
        jnp.exp(lm - m_new), axis=0, keepdims=True
    )
    m_ref[...] = m_new
    ll_ref[...] = ll_ref[...] + jnp.sum(
        jnp.where(row == lbl_ref[...], lm, 0.0), axis=0, keepdims=True
    )

    @pl.when(j == _NV - 1)
    def _finish():
        lse = m_ref[...] + jnp.log(s_ref[...])
        loss_ref[...] = (jnp.sum(lse - ll_ref[...]) * (1.0 / _B)).reshape(1, 1)


_tc_call = pl.pallas_call(
    _tc_body,
    grid=(_NV,),
    in_specs=[
        pl.BlockSpec((_B, _HID), lambda j: (0, 0)),
        pl.BlockSpec((_HID, _VBLK), lambda j: (0, j)),
        pl.BlockSpec((1, _VBLK), lambda j: (0, j)),
        pl.BlockSpec((1, _B), lambda j: (0, 0)),
    ],
    out_specs=[
        pl.BlockSpec((_VBLK, _B), lambda j: (j, 0)),
        pl.BlockSpec((1, 1), lambda j: (0, 0)),
    ],
    out_shape=[
        jax.ShapeDtypeStruct((_VOCAB, _B), jnp.float32),
        jax.ShapeDtypeStruct((1, 1), jnp.float32),
    ],
    scratch_shapes=[
        pltpu.VMEM((1, _B), jnp.float32),
        pltpu.VMEM((1, _B), jnp.float32),
        pltpu.VMEM((1, _B), jnp.float32),
    ],
    compiler_params=pltpu.CompilerParams(
        dimension_semantics=("arbitrary",),
    ),
)


def kernel(input_ids, labels, emb, W, b):
    ids = input_ids.astype(jnp.int32)
    x = _get_sc_gather()(emb, ids)
    b2 = b.reshape(1, _VOCAB)
    lbl = labels.astype(jnp.int32).reshape(1, _B)
    logits_t, loss = _tc_call(x, W, b2, lbl)
    return loss[0, 0], logits_t.T


# speedup vs baseline: 2.2777x; 1.0069x over previous
import functools

import jax
import jax.numpy as jnp
from jax import lax
from jax.experimental import pallas as pl
from jax.experimental.pallas import tpu as pltpu
from jax.experimental.pallas import tpu_sc as plsc

_VOCAB = 100000
_HID = 64
_B = 1024
_VBLK = 2048
_NV = (_VOCAB + _VBLK - 1) // _VBLK

_RBLK = 1024
_NR = 49
_SPLIT = _RBLK * _NR


def _relayout_body(lo_ref, hi_ref, out_ref):
    ii = lax.broadcasted_iota(jnp.int32, (_HID, _HID), 0)
    jj = lax.broadcasted_iota(jnp.int32, (_HID, _HID), 1)
    ident = jnp.where(ii == jj, 1.0, 0.0).astype(jnp.float32)
    dims = (((0,), (0,)), ((), ()))
    t_lo = lax.dot_general(lo_ref[...], ident, dims,
                           preferred_element_type=jnp.float32)
    t_hi = lax.dot_general(hi_ref[...], ident, dims,
                           preferred_element_type=jnp.float32)
    out_ref[...] = jnp.concatenate([t_lo, t_hi], axis=1)


_relayout_call = pl.pallas_call(
    _relayout_body,
    grid=(_NR,),
    in_specs=[
        pl.BlockSpec((_HID, _RBLK), lambda j: (0, j)),
        pl.BlockSpec((_HID, _RBLK), lambda j: (0, j + _NR)),
    ],
    out_specs=pl.BlockSpec((_RBLK, 2 * _HID), lambda j: (j, 0)),
    out_shape=jax.ShapeDtypeStruct((_SPLIT, 2 * _HID), jnp.float32),
)


def _build_sc_gather():
    info = plsc.get_sparse_core_info()
    nc, ns = info.num_cores, info.num_subcores
    nw = nc * ns
    b_per_w = _B // nw
    mesh = plsc.VectorSubcoreMesh(core_axis_name="c", subcore_axis_name="s")

    @functools.partial(
        pl.kernel,
        mesh=mesh,
        out_type=jax.ShapeDtypeStruct((_B, 2 * _HID), jnp.float32),
        scratch_types=[
            pltpu.VMEM((b_per_w,), jnp.int32),
            pltpu.VMEM((b_per_w, 2 * _HID), jnp.float32),
            pltpu.SemaphoreType.DMA,
        ],
    )
    def gather_rows(table_hbm, idx_hbm, out_hbm, idx_v, rows_v, sem):
        wid = lax.axis_index("s") * nc + lax.axis_index("c")
        base = wid * b_per_w
        pltpu.sync_copy(idx_hbm.at[pl.ds(base, b_per_w)], idx_v)
        pltpu.async_copy(table_hbm.at[idx_v], rows_v, sem).wait()
        pltpu.sync_copy(rows_v, out_hbm.at[pl.ds(base, b_per_w)])

    return gather_rows


_sc_gather_cache = []


def _get_sc_gather():
    if not _sc_gather_cache:
        _sc_gather_cache.append(_build_sc_gather())
    return _sc_gather_cache[0]


def _tc_body(x_ref, w_ref, b_ref, lbl_ref, par_ref, logits_ref, loss_ref,
             m_ref, s_ref, ll_ref):
    j = pl.program_id(0)

    @pl.when(j == 0)
    def _init():
        m_ref[...] = jnp.full((1, _B), -jnp.inf, jnp.float32)
        s_ref[...] = jnp.zeros((1, _B), jnp.float32)
        ll_ref[...] = jnp.zeros((1, _B), jnp.float32)

    x_sel = jnp.where(par_ref[...] != 0, x_ref[:, _HID:2 * _HID], x_ref[:, :_HID])
    x_aug = jnp.concatenate([x_sel, jnp.ones((_B, 1), jnp.float32)], axis=1)
    w_aug = jnp.concatenate([w_ref[...], b_ref[...]], axis=0)
    logits = lax.dot_general(
        w_aug,
        x_aug,
        (((0,), (1,)), ((), ())),
        preferred_element_type=jnp.float32,
    )
    logits_ref[...] = logits

    row = j * _VBLK + lax.broadcasted_iota(jnp.int32, (_VBLK, _B), 0)
    lm = jnp.where(row < _VOCAB, logits, -jnp.inf)
    m_old = m_ref[...]
    m_new = jnp.maximum(m_old, jnp.max(lm, axis=0, keepdims=True))
    s_ref[...] = s_ref[...] * jnp.exp(m_old - m_new) + jnp.sum(
        jnp.exp(lm - m_new), axis=0, keepdims=True
    )
    m_ref[...] = m_new
    ll_ref[...] = ll_ref[...] + jnp.sum(
        jnp.where(row == lbl_ref[...], lm, 0.0), axis=0, keepdims=True
    )

    @pl.when(j == _NV - 1)
    def _finish():
        lse = m_ref[...] + jnp.log(s_ref[...])
        loss_ref[...] = (jnp.sum(lse - ll_ref[...]) * (1.0 / _B)).reshape(1, 1)


_tc_call = pl.pallas_call(
    _tc_body,
    grid=(_NV,),
    in_specs=[
        pl.BlockSpec((_B, 2 * _HID), lambda j: (0, 0)),
        pl.BlockSpec((_HID, _VBLK), lambda j: (0, j)),
        pl.BlockSpec((1, _VBLK), lambda j: (0, j)),
        pl.BlockSpec((1, _B), lambda j: (0, 0)),
        pl.BlockSpec((_B, 1), lambda j: (0, 0)),
    ],
    out_specs=[
        pl.BlockSpec((_VBLK, _B), lambda j: (j, 0)),
        pl.BlockSpec((1, 1), lambda j: (0, 0)),
    ],
    out_shape=[
        jax.ShapeDtypeStruct((_VOCAB, _B), jnp.float32),
        jax.ShapeDtypeStruct((1, 1), jnp.float32),
    ],
    scratch_shapes=[
        pltpu.VMEM((1, _B), jnp.float32),
        pltpu.VMEM((1, _B), jnp.float32),
        pltpu.VMEM((1, _B), jnp.float32),
    ],
    compiler_params=pltpu.CompilerParams(
        dimension_semantics=("arbitrary",),
    ),
)


def kernel(input_ids, labels, emb, W, b):
    ids = input_ids.astype(jnp.int32)
    table = _relayout_call(emb.T, emb.T)
    x = _get_sc_gather()(table, jnp.where(ids >= _SPLIT, ids - _SPLIT, ids))
    par = (ids >= _SPLIT).astype(jnp.int32).reshape(_B, 1)
    b2 = b.reshape(1, _VOCAB)
    lbl = labels.astype(jnp.int32).reshape(1, _B)
    logits_t, loss = _tc_call(x, W, b2, lbl, par)
    return loss[0, 0], logits_t.T


# speedup vs baseline: 2.5709x; 1.1287x over previous
import functools

import jax
import jax.numpy as jnp
from jax import lax
from jax.experimental import pallas as pl
from jax.experimental.pallas import tpu as pltpu
from jax.experimental.pallas import tpu_sc as plsc

_VOCAB = 100000
_HID = 64
_B = 1024
_VBLK = 2048
_NV = (_VOCAB + _VBLK - 1) // _VBLK

_RBLK = 3584
_NR = 14
_SPLIT = _RBLK * _NR


def _relayout_body(lo_ref, hi_ref, out_ref):
    ii = lax.broadcasted_iota(jnp.int32, (_HID, _HID), 0)
    jj = lax.broadcasted_iota(jnp.int32, (_HID, _HID), 1)
    ident = jnp.where(ii == jj, 1.0, 0.0).astype(jnp.float32)
    dims = (((0,), (0,)), ((), ()))
    t_lo = lax.dot_general(lo_ref[...], ident, dims,
                           preferred_element_type=jnp.float32)
    t_hi = lax.dot_general(hi_ref[...], ident, dims,
                           preferred_element_type=jnp.float32)
    out_ref[...] = jnp.concatenate([t_lo, t_hi], axis=1)


_relayout_call = pl.pallas_call(
    _relayout_body,
    grid=(_NR,),
    in_specs=[
        pl.BlockSpec((_HID, _RBLK), lambda j: (0, j)),
        pl.BlockSpec((_HID, _RBLK), lambda j: (0, j + _NR)),
    ],
    out_specs=pl.BlockSpec((_RBLK, 2 * _HID), lambda j: (j, 0)),
    out_shape=jax.ShapeDtypeStruct((_SPLIT, 2 * _HID), jnp.float32),
)


def _build_sc_gather():
    info = plsc.get_sparse_core_info()
    nc, ns = info.num_cores, info.num_subcores
    nw = nc * ns
    b_per_w = _B // nw
    mesh = plsc.VectorSubcoreMesh(core_axis_name="c", subcore_axis_name="s")

    @functools.partial(
        pl.kernel,
        mesh=mesh,
        out_type=jax.ShapeDtypeStruct((_B, 2 * _HID), jnp.float32),
        scratch_types=[
            pltpu.VMEM((b_per_w,), jnp.int32),
            pltpu.VMEM((b_per_w, 2 * _HID), jnp.float32),
            pltpu.SemaphoreType.DMA,
        ],
    )
    def gather_rows(table_hbm, idx_hbm, out_hbm, idx_v, rows_v, sem):
        wid = lax.axis_index("s") * nc + lax.axis_index("c")
        base = wid * b_per_w
        pltpu.sync_copy(idx_hbm.at[pl.ds(base, b_per_w)], idx_v)
        pltpu.async_copy(table_hbm.at[idx_v], rows_v, sem).wait()
        pltpu.sync_copy(rows_v, out_hbm.at[pl.ds(base, b_per_w)])

    return gather_rows


_sc_gather_cache = []


def _get_sc_gather():
    if not _sc_gather_cache:
        _sc_gather_cache.append(_build_sc_gather())
    return _sc_gather_cache[0]


def _tc_body(x_ref, w_ref, b_ref, lbl_ref, logits_ref, loss_ref,
             m_ref, s_ref, ll_ref):
    j = pl.program_id(0)

    @pl.when(j == 0)
    def _init():
        m_ref[...] = jnp.full((1, _B), -jnp.inf, jnp.float32)
        s_ref[...] = jnp.zeros((1, _B), jnp.float32)
        ll_ref[...] = jnp.zeros((1, _B), jnp.float32)

    w_aug = jnp.concatenate([w_ref[...], b_ref[...]], axis=0)
    logits = lax.dot_general(
        w_aug,
        x_ref[...],
        (((0,), (1,)), ((), ())),
        preferred_element_type=jnp.float32,
    )
    logits_ref[...] = logits

    row = j * _VBLK + lax.broadcasted_iota(jnp.int32, (_VBLK, _B), 0)
    lm = jnp.where(row < _VOCAB, logits, -jnp.inf)
    m_old = m_ref[...]
    m_new = jnp.maximum(m_old, jnp.max(lm, axis=0, keepdims=True))
    s_ref[...] = s_ref[...] * jnp.exp(m_old - m_new) + jnp.sum(
        jnp.exp(lm - m_new), axis=0, keepdims=True
    )
    m_ref[...] = m_new
    ll_ref[...] = ll_ref[...] + jnp.sum(
        jnp.where(row == lbl_ref[...], lm, 0.0), axis=0, keepdims=True
    )

    @pl.when(j == _NV - 1)
    def _finish():
        lse = m_ref[...] + jnp.log(s_ref[...])
        loss_ref[...] = (jnp.sum(lse - ll_ref[...]) * (1.0 / _B)).reshape(1, 1)


_tc_call = pl.pallas_call(
    _tc_body,
    grid=(_NV,),
    in_specs=[
        pl.BlockSpec((_B, _HID + 1), lambda j: (0, 0)),
        pl.BlockSpec((_HID, _VBLK), lambda j: (0, j)),
        pl.BlockSpec((1, _VBLK), lambda j: (0, j)),
        pl.BlockSpec((1, _B), lambda j: (0, 0)),
    ],
    out_specs=[
        pl.BlockSpec((_VBLK, _B), lambda j: (j, 0)),
        pl.BlockSpec((1, 1), lambda j: (0, 0)),
    ],
    out_shape=[
        jax.ShapeDtypeStruct((_VOCAB, _B), jnp.float32),
        jax.ShapeDtypeStruct((1, 1), jnp.float32),
    ],
    scratch_shapes=[
        pltpu.VMEM((1, _B), jnp.float32),
        pltpu.VMEM((1, _B), jnp.float32),
        pltpu.VMEM((1, _B), jnp.float32),
    ],
    compiler_params=pltpu.CompilerParams(
        dimension_semantics=("arbitrary",),
    ),
)


def kernel(input_ids, labels, emb, W, b):
    ids = input_ids.astype(jnp.int32)
    table = _relayout_call(emb.T, emb.T)
    x = _get_sc_gather()(table, jnp.where(ids >= _SPLIT, ids - _SPLIT, ids))
    par = (ids >= _SPLIT).reshape(_B, 1)
    x_sel = jnp.where(par, x[:, _HID:2 * _HID], x[:, :_HID])
    x_aug = jnp.concatenate([x_sel, jnp.ones((_B, 1), jnp.float32)], axis=1)
    b2 = b.reshape(1, _VOCAB)
    lbl = labels.astype(jnp.int32).reshape(1, _B)
    logits_t, loss = _tc_call(x_aug, W, b2, lbl)
    return loss[0, 0], logits_t.T


# speedup vs baseline: 2.6536x; 1.0322x over previous
import functools

import jax
import jax.numpy as jnp
from jax import lax
from jax.experimental import pallas as pl
from jax.experimental.pallas import tpu as pltpu
from jax.experimental.pallas import tpu_sc as plsc

_VOCAB = 100000
_HID = 64
_B = 1024
_VBLK = 4096
_NV = (_VOCAB + _VBLK - 1) // _VBLK

_RBLK = 7168
_NR = 7
_SPLIT = _RBLK * _NR


def _relayout_body(lo_ref, hi_ref, out_ref):
    ii = lax.broadcasted_iota(jnp.int32, (_HID, _HID), 0)
    jj = lax.broadcasted_iota(jnp.int32, (_HID, _HID), 1)
    ident = jnp.where(ii == jj, 1.0, 0.0).astype(jnp.float32)
    dims = (((0,), (0,)), ((), ()))
    t_lo = lax.dot_general(lo_ref[...], ident, dims,
                           preferred_element_type=jnp.float32)
    t_hi = lax.dot_general(hi_ref[...], ident, dims,
                           preferred_element_type=jnp.float32)
    out_ref[...] = jnp.concatenate([t_lo, t_hi], axis=1)


_relayout_call = pl.pallas_call(
    _relayout_body,
    grid=(_NR,),
    in_specs=[
        pl.BlockSpec((_HID, _RBLK), lambda j: (0, j)),
        pl.BlockSpec((_HID, _RBLK), lambda j: (0, j + _NR)),
    ],
    out_specs=pl.BlockSpec((_RBLK, 2 * _HID), lambda j: (j, 0)),
    out_shape=jax.ShapeDtypeStruct((_SPLIT, 2 * _HID), jnp.float32),
)


def _build_sc_gather():
    info = plsc.get_sparse_core_info()
    nc, ns = info.num_cores, info.num_subcores
    nw = nc * ns
    b_per_w = _B // nw
    mesh = plsc.VectorSubcoreMesh(core_axis_name="c", subcore_axis_name="s")

    @functools.partial(
        pl.kernel,
        mesh=mesh,
        out_type=jax.ShapeDtypeStruct((_B, 2 * _HID), jnp.float32),
        scratch_types=[
            pltpu.VMEM((b_per_w,), jnp.int32),
            pltpu.VMEM((b_per_w, 2 * _HID), jnp.float32),
            pltpu.SemaphoreType.DMA,
        ],
    )
    def gather_rows(table_hbm, idx_hbm, out_hbm, idx_v, rows_v, sem):
        wid = lax.axis_index("s") * nc + lax.axis_index("c")
        base = wid * b_per_w
        pltpu.sync_copy(idx_hbm.at[pl.ds(base, b_per_w)], idx_v)
        pltpu.async_copy(table_hbm.at[idx_v], rows_v, sem).wait()
        pltpu.sync_copy(rows_v, out_hbm.at[pl.ds(base, b_per_w)])

    return gather_rows


_sc_gather_cache = []


def _get_sc_gather():
    if not _sc_gather_cache:
        _sc_gather_cache.append(_build_sc_gather())
    return _sc_gather_cache[0]


def _tc_body(x_ref, w_ref, b_ref, lbl_ref, logits_ref, loss_ref,
             m_ref, s_ref, ll_ref):
    j = pl.program_id(0)

    @pl.when(j == 0)
    def _init():
        m_ref[...] = jnp.full((1, _B), -jnp.inf, jnp.float32)
        s_ref[...] = jnp.zeros((1, _B), jnp.float32)
        ll_ref[...] = jnp.zeros((1, _B), jnp.float32)

    w_aug = jnp.concatenate([w_ref[...], b_ref[...]], axis=0)
    logits = lax.dot_general(
        w_aug,
        x_ref[...],
        (((0,), (1,)), ((), ())),
        preferred_element_type=jnp.float32,
    )
    logits_ref[...] = logits

    row = j * _VBLK + lax.broadcasted_iota(jnp.int32, (_VBLK, _B), 0)
    lm = jnp.where(row < _VOCAB, logits, -jnp.inf)
    m_old = m_ref[...]
    m_new = jnp.maximum(m_old, jnp.max(lm, axis=0, keepdims=True))
    s_ref[...] = s_ref[...] * jnp.exp(m_old - m_new) + jnp.sum(
        jnp.exp(lm - m_new), axis=0, keepdims=True
    )
    m_ref[...] = m_new
    ll_ref[...] = ll_ref[...] + jnp.sum(
        jnp.where(row == lbl_ref[...], lm, 0.0), axis=0, keepdims=True
    )

    @pl.when(j == _NV - 1)
    def _finish():
        lse = m_ref[...] + jnp.log(s_ref[...])
        loss_ref[...] = (jnp.sum(lse - ll_ref[...]) * (1.0 / _B)).reshape(1, 1)


_tc_call = pl.pallas_call(
    _tc_body,
    grid=(_NV,),
    in_specs=[
        pl.BlockSpec((_B, _HID + 1), lambda j: (0, 0)),
        pl.BlockSpec((_HID, _VBLK), lambda j: (0, j)),
        pl.BlockSpec((1, _VBLK), lambda j: (0, j)),
        pl.BlockSpec((1, _B), lambda j: (0, 0)),
    ],
    out_specs=[
        pl.BlockSpec((_VBLK, _B), lambda j: (j, 0)),
        pl.BlockSpec((1, 1), lambda j: (0, 0)),
    ],
    out_shape=[
        jax.ShapeDtypeStruct((_VOCAB, _B), jnp.float32),
        jax.ShapeDtypeStruct((1, 1), jnp.float32),
    ],
    scratch_shapes=[
        pltpu.VMEM((1, _B), jnp.float32),
        pltpu.VMEM((1, _B), jnp.float32),
        pltpu.VMEM((1, _B), jnp.float32),
    ],
    compiler_params=pltpu.CompilerParams(
        dimension_semantics=("arbitrary",),
    ),
)


def kernel(input_ids, labels, emb, W, b):
    ids = input_ids.astype(jnp.int32)
    table = _relayout_call(emb.T, emb.T)
    x = _get_sc_gather()(table, jnp.where(ids >= _SPLIT, ids - _SPLIT, ids))
    par = (ids >= _SPLIT).reshape(_B, 1)
    x_sel = jnp.where(par, x[:, _HID:2 * _HID], x[:, :_HID])
    x_aug = jnp.concatenate([x_sel, jnp.ones((_B, 1), jnp.float32)], axis=1)
    b2 = b.reshape(1, _VOCAB)
    lbl = labels.astype(jnp.int32).reshape(1, _B)
    logits_t, loss = _tc_call(x_aug, W, b2, lbl)
    return loss[0, 0], logits_t.T
